# baseline (speedup 1.0000x reference)
import functools

import jax
import jax.numpy as jnp
from jax import lax
from jax.experimental import pallas as pl
from jax.experimental.pallas import tpu as pltpu
from jax.experimental.pallas import tpu_sc as plsc

_E = 8
_K = 2
_D = 1024
_F = 4096
_BLK = 256
_FF = 4096
_NF = _F // _FF
_LANES = 128


def _router_body(x_ref, wr_ref, idx_ref, prob_ref):
    x = x_ref[...]
    wr = wr_ref[...]
    logits = jnp.dot(x, wr, preferred_element_type=jnp.float32)
    lane = lax.broadcasted_iota(jnp.int32, logits.shape, 1)
    neg = jnp.float32(-1e30)
    logits = jnp.where(lane < _E, logits, neg)
    m1 = jnp.max(logits, axis=1, keepdims=True)
    a1 = jnp.min(jnp.where(logits == m1, lane, _LANES), axis=1, keepdims=True)
    l2 = jnp.where(lane == a1, neg, logits)
    m2 = jnp.max(l2, axis=1, keepdims=True)
    a2 = jnp.min(jnp.where(l2 == m2, lane, _LANES), axis=1, keepdims=True)
    e2 = jnp.exp(m2 - m1)
    p1 = 1.0 / (1.0 + e2)
    p2 = e2 / (1.0 + e2)
    idx_ref[...] = jnp.where(lane == 0, a1, jnp.where(lane == 1, a2, 0))
    prob_ref[...] = jnp.where(lane == 0, p1, jnp.where(lane == 1, p2, 0.0))


def _router(x, wr_pad):
    n = x.shape[0]
    t = 512
    idx, prob = pl.pallas_call(
        _router_body,
        grid=(n // t,),
        in_specs=[
            pl.BlockSpec((t, _D), lambda i: (i, 0)),
            pl.BlockSpec((_D, _LANES), lambda i: (0, 0)),
        ],
        out_specs=[
            pl.BlockSpec((t, _LANES), lambda i: (i, 0)),
            pl.BlockSpec((t, _LANES), lambda i: (i, 0)),
        ],
        out_shape=[
            jax.ShapeDtypeStruct((n, _LANES), jnp.int32),
            jax.ShapeDtypeStruct((n, _LANES), jnp.float32),
        ],
    )(x, wr_pad)
    return idx[:, :_K], prob[:, :_K]


def _plan(topi, probs, n):
    p_total = _K * n + _E * _BLK
    e_pairs = topi.reshape(-1)
    w_pairs = probs.reshape(-1)
    oh = (e_pairs[:, None] == jnp.arange(_E, dtype=jnp.int32)[None, :]).astype(jnp.int32)
    counts = jnp.sum(oh, axis=0)
    rank = jnp.sum((jnp.cumsum(oh, axis=0) - oh) * oh, axis=1)
    padded = ((counts + _BLK - 1) // _BLK) * _BLK
    ends = jnp.cumsum(padded)
    starts = ends - padded
    dest = starts[e_pairs] + rank
    row_token = jnp.zeros((p_total,), jnp.int32).at[dest].set(
        jnp.arange(_K * n, dtype=jnp.int32) // _K)
    row_w = jnp.zeros((p_total,), jnp.float32).at[dest].set(w_pairs)
    bstart = jnp.arange(p_total // _BLK, dtype=jnp.int32) * _BLK
    block_expert = jnp.minimum(
        jnp.sum((bstart[:, None] >= ends[None, :]).astype(jnp.int32), axis=1),
        _E - 1).astype(jnp.int32)
    return row_token, row_w, block_expert, dest.astype(jnp.int32)


def _sc_gather(row_token, x_flat):
    p_total = row_token.shape[0]
    info = plsc.get_sparse_core_info()
    nw = info.num_cores * info.num_subcores
    per_w = p_total // nw
    ch = 64
    n_ch = per_w // ch
    mesh = plsc.VectorSubcoreMesh(core_axis_name="c", subcore_axis_name="s")

    @functools.partial(
        pl.kernel,
        mesh=mesh,
        out_type=jax.ShapeDtypeStruct((p_total, _D), jnp.float32),
        scratch_types=[
            pltpu.VMEM((ch,), jnp.int32),
            pltpu.VMEM((ch, _D), jnp.float32),
            pltpu.SemaphoreType.DMA,
        ],
    )
    def k(tok_hbm, x_hbm, out_hbm, idx_v, rows_v, sem):
        wid = lax.axis_index("s") * info.num_cores + lax.axis_index("c")
        base = wid * per_w

        def body(c, _):
            off = base + c * ch
            pltpu.sync_copy(tok_hbm.at[pl.ds(off, ch)], idx_v)
            pltpu.async_copy(x_hbm.at[idx_v], rows_v, sem).wait()
            pltpu.sync_copy(rows_v, out_hbm.at[pl.ds(off, ch)])
            return 0

        lax.fori_loop(0, n_ch, body, 0)

    return k(row_token, x_flat)


def _swiglu_part(x_ref, w_ref, wg_ref, wu_ref, wd_ref):
    x = x_ref[...].astype(jnp.bfloat16)
    g = jnp.dot(x, wg_ref[0], preferred_element_type=jnp.float32)
    u = jnp.dot(x, wu_ref[0], preferred_element_type=jnp.float32)
    h = (g * jax.nn.sigmoid(g) * u).astype(jnp.bfloat16)
    part = jnp.dot(h, wd_ref[0], preferred_element_type=jnp.float32)
    return part * w_ref[:, :1]


def _gemm_body_first(be_ref, x_ref, w_ref, wg_ref, wu_ref, wd_ref, y_ref):
    y_ref[...] = _swiglu_part(x_ref, w_ref, wg_ref, wu_ref, wd_ref)


def _gemm_body_acc(be_ref, y_in_ref, x_ref, w_ref, wg_ref, wu_ref, wd_ref,
                   y_ref):
    y_ref[...] = y_in_ref[...] + _swiglu_part(x_ref, w_ref, wg_ref, wu_ref,
                                              wd_ref)


def _gemm(block_expert, x_sorted, w128, wg, wu, wd):
    p_total = x_sorted.shape[0]
    nb = p_total // _BLK
    row_specs = [
        pl.BlockSpec((_BLK, _D), lambda b, be: (b, 0)),
        pl.BlockSpec((_BLK, _LANES), lambda b, be: (b, 0)),
    ]
    out_spec = pl.BlockSpec((_BLK, _D), lambda b, be: (b, 0))
    out_shape = jax.ShapeDtypeStruct((p_total, _D), jnp.float32)
    y = None
    for f in range(_NF):
        w_specs = [
            pl.BlockSpec((1, _D, _FF), lambda b, be, f=f: (be[b], 0, f)),
            pl.BlockSpec((1, _D, _FF), lambda b, be, f=f: (be[b], 0, f)),
            pl.BlockSpec((1, _FF, _D), lambda b, be, f=f: (be[b], f, 0)),
        ]
        if f == 0:
            y = pl.pallas_call(
                _gemm_body_first,
                grid_spec=pltpu.PrefetchScalarGridSpec(
                    num_scalar_prefetch=1,
                    grid=(nb,),
                    in_specs=row_specs + w_specs,
                    out_specs=out_spec,
                ),
                out_shape=out_shape,
            )(block_expert, x_sorted, w128, wg, wu, wd)
        else:
            y = pl.pallas_call(
                _gemm_body_acc,
                grid_spec=pltpu.PrefetchScalarGridSpec(
                    num_scalar_prefetch=1,
                    grid=(nb,),
                    in_specs=[out_spec] + row_specs + w_specs,
                    out_specs=out_spec,
                ),
                out_shape=out_shape,
                input_output_aliases={1: 0},
            )(block_expert, y, x_sorted, w128, wg, wu, wd)
    return y


def _sc_combine(pos_flat, yw):
    n = pos_flat.shape[0] // _K
    info = plsc.get_sparse_core_info()
    nw = info.num_cores * info.num_subcores
    per_w = n // nw
    ch = 16
    n_ch = per_w // ch
    mesh = plsc.VectorSubcoreMesh(core_axis_name="c", subcore_axis_name="s")

    @functools.partial(
        pl.kernel,
        mesh=mesh,
        out_type=jax.ShapeDtypeStruct((n, _D), jnp.float32),
        scratch_types=[
            pltpu.VMEM((_K * ch,), jnp.int32),
            pltpu.VMEM((_K * ch, _D), jnp.float32),
            pltpu.VMEM((ch, _D), jnp.float32),
            pltpu.SemaphoreType.DMA,
        ],
    )
    def k(pos_hbm, yw_hbm, out_hbm, idx_v, rows_v, out_v, sem):
        wid = lax.axis_index("s") * info.num_cores + lax.axis_index("c")
        base = wid * per_w

        def body(c, _):
            off = base + c * ch
            pltpu.sync_copy(pos_hbm.at[pl.ds(_K * off, _K * ch)], idx_v)
            pltpu.async_copy(yw_hbm.at[idx_v], rows_v, sem).wait()
            for j in range(ch):
                for t in range(_D // 16):
                    sl = pl.ds(t * 16, 16)
                    out_v[j, sl] = rows_v[_K * j, sl] + rows_v[_K * j + 1, sl]
            pltpu.sync_copy(out_v, out_hbm.at[pl.ds(off, ch)])
            return 0

        lax.fori_loop(0, n_ch, body, 0)

    return k(pos_flat, yw)


def kernel(input, W_router, W_gate, W_up, W_down):
    b, s, d = input.shape
    n = b * s
    x = input.reshape(n, d)
    wr_pad = jnp.zeros((d, _LANES), jnp.float32).at[:, :_E].set(W_router)
    topi, probs = _router(x, wr_pad)
    row_token, row_w, block_expert, pos = _plan(topi, probs, n)
    x_sorted = _sc_gather(row_token, x)
    w128 = jnp.broadcast_to(row_w[:, None], (row_w.shape[0], _LANES))
    yw = _gemm(block_expert, x_sorted, w128,
               W_gate.astype(jnp.bfloat16), W_up.astype(jnp.bfloat16),
               W_down.astype(jnp.bfloat16))
    out = _sc_combine(pos, yw)
    return out.reshape(b, s, d)


# speedup vs baseline: 1.4470x; 1.4470x over previous
import functools

import jax
import jax.numpy as jnp
from jax import lax
from jax.experimental import pallas as pl
from jax.experimental.pallas import tpu as pltpu
from jax.experimental.pallas import tpu_sc as plsc

_E = 8
_K = 2
_D = 1024
_F = 4096
_BLK = 256
_FF = 2048
_NF = _F // _FF
_LANES = 128


def _router_body(x_ref, wr_ref, idx_ref, prob_ref):
    x = x_ref[...]
    wr = wr_ref[...]
    logits = jnp.dot(x, wr, preferred_element_type=jnp.float32)
    lane = lax.broadcasted_iota(jnp.int32, logits.shape, 1)
    neg = jnp.float32(-1e30)
    logits = jnp.where(lane < _E, logits, neg)
    m1 = jnp.max(logits, axis=1, keepdims=True)
    a1 = jnp.min(jnp.where(logits == m1, lane, _LANES), axis=1, keepdims=True)
    l2 = jnp.where(lane == a1, neg, logits)
    m2 = jnp.max(l2, axis=1, keepdims=True)
    a2 = jnp.min(jnp.where(l2 == m2, lane, _LANES), axis=1, keepdims=True)
    e2 = jnp.exp(m2 - m1)
    p1 = 1.0 / (1.0 + e2)
    p2 = e2 / (1.0 + e2)
    idx_ref[...] = jnp.where(lane == 0, a1, jnp.where(lane == 1, a2, 0))
    prob_ref[...] = jnp.where(lane == 0, p1, jnp.where(lane == 1, p2, 0.0))


def _router(x, wr_pad):
    n = x.shape[0]
    t = 512
    idx, prob = pl.pallas_call(
        _router_body,
        grid=(n // t,),
        in_specs=[
            pl.BlockSpec((t, _D), lambda i: (i, 0)),
            pl.BlockSpec((_D, _LANES), lambda i: (0, 0)),
        ],
        out_specs=[
            pl.BlockSpec((t, _LANES), lambda i: (i, 0)),
            pl.BlockSpec((t, _LANES), lambda i: (i, 0)),
        ],
        out_shape=[
            jax.ShapeDtypeStruct((n, _LANES), jnp.int32),
            jax.ShapeDtypeStruct((n, _LANES), jnp.float32),
        ],
    )(x, wr_pad)
    return idx[:, :_K], prob[:, :_K]


def _plan(topi, probs, n):
    p_total = _K * n + _E * _BLK
    e_pairs = topi.reshape(-1)
    w_pairs = probs.reshape(-1)
    oh = (e_pairs[:, None] == jnp.arange(_E, dtype=jnp.int32)[None, :]).astype(jnp.int32)
    counts = jnp.sum(oh, axis=0)
    rank = jnp.sum((jnp.cumsum(oh, axis=0) - oh) * oh, axis=1)
    padded = ((counts + _BLK - 1) // _BLK) * _BLK
    ends = jnp.cumsum(padded)
    starts = ends - padded
    dest = starts[e_pairs] + rank
    row_w = jnp.zeros((p_total,), jnp.float32).at[dest].set(w_pairs)
    bstart = jnp.arange(p_total // _BLK, dtype=jnp.int32) * _BLK
    block_expert = jnp.minimum(
        jnp.sum((bstart[:, None] >= ends[None, :]).astype(jnp.int32), axis=1),
        _E - 1).astype(jnp.int32)
    return row_w, block_expert, dest.astype(jnp.int32)


def _sc_dispatch(dest3, x_flat, p_total):
    info = plsc.get_sparse_core_info()
    nw = info.num_cores * info.num_subcores
    n_ch, ch = dest3.shape[1], dest3.shape[2]
    per_w = n_ch * ch
    mesh = plsc.VectorSubcoreMesh(core_axis_name="c", subcore_axis_name="s")

    @functools.partial(
        pl.kernel,
        mesh=mesh,
        out_type=jax.ShapeDtypeStruct((p_total, _D), jnp.float32),
        scratch_types=[
            pltpu.VMEM((ch,), jnp.int32),
            pltpu.VMEM((ch,), jnp.int32),
            pltpu.VMEM((ch,), jnp.int32),
            pltpu.VMEM((ch,), jnp.int32),
            pltpu.VMEM((ch, _D), jnp.float32),
            pltpu.VMEM((ch, _D), jnp.float32),
            pltpu.SemaphoreType.DMA,
            pltpu.SemaphoreType.DMA,
            pltpu.SemaphoreType.DMA,
            pltpu.SemaphoreType.DMA,
        ],
    )
    def k(dest_hbm, x_hbm, out_hbm, i0, i1, d0, d1, r0, r1, g0, g1, s0, s1):
        wid = lax.axis_index("s") * info.num_cores + lax.axis_index("c")
        tbase = wid * (per_w // _K)
        ibuf, dbuf, rbuf = [i0, i1], [d0, d1], [r0, r1]
        gsem, ssem = [g0, g1], [s0, s1]
        iota = lax.broadcasted_iota(jnp.int32, (16,), 0)

        def fire_gather(c, s):
            for t in range(ch // 16):
                ibuf[s][pl.ds(16 * t, 16)] = (
                    tbase + c * (ch // _K) + ((iota + 16 * t) >> 1))
            return pltpu.async_copy(x_hbm.at[ibuf[s]], rbuf[s], gsem[s])

        pend_g = [fire_gather(0, 0), None]
        pend_s = [None, None]
        for c in range(n_ch):
            s = c % 2
            o = (c + 1) % 2
            if c + 1 < n_ch:
                if pend_s[o] is not None:
                    pend_s[o].wait()
                pend_g[o] = fire_gather(c + 1, o)
            pend_g[s].wait()
            pltpu.sync_copy(dest_hbm.at[wid, c], dbuf[s])
            pend_s[s] = pltpu.async_copy(rbuf[s], out_hbm.at[dbuf[s]], ssem[s])
        for s in range(2):
            if pend_s[s] is not None:
                pend_s[s].wait()

    return k(dest3, x_flat)


def _swiglu_part(x_ref, w_ref, wg_ref, wu_ref, wd_ref):
    x = x_ref[...].astype(jnp.bfloat16)
    g = jnp.dot(x, wg_ref[0].astype(jnp.bfloat16),
                preferred_element_type=jnp.float32)
    u = jnp.dot(x, wu_ref[0].astype(jnp.bfloat16),
                preferred_element_type=jnp.float32)
    h = (g * jax.nn.sigmoid(g) * u).astype(jnp.bfloat16)
    part = jnp.dot(h, wd_ref[0].astype(jnp.bfloat16),
                   preferred_element_type=jnp.float32)
    return part * w_ref[:, :1]


def _gemm_body_first(be_ref, x_ref, w_ref, wg_ref, wu_ref, wd_ref, y_ref):
    y_ref[...] = _swiglu_part(x_ref, w_ref, wg_ref, wu_ref, wd_ref)


def _gemm_body_acc(be_ref, y_in_ref, x_ref, w_ref, wg_ref, wu_ref, wd_ref,
                   y_ref):
    y_ref[...] = y_in_ref[...] + _swiglu_part(x_ref, w_ref, wg_ref, wu_ref,
                                              wd_ref)


def _gemm(block_expert, x_sorted, w128, wg, wu, wd):
    p_total = x_sorted.shape[0]
    nb = p_total // _BLK
    row_specs = [
        pl.BlockSpec((_BLK, _D), lambda b, be: (b, 0)),
        pl.BlockSpec((_BLK, _LANES), lambda b, be: (b, 0)),
    ]
    out_spec = pl.BlockSpec((_BLK, _D), lambda b, be: (b, 0))
    out_shape = jax.ShapeDtypeStruct((p_total, _D), jnp.float32)
    y = None
    for f in range(_NF):
        w_specs = [
            pl.BlockSpec((1, _D, _FF), lambda b, be, f=f: (be[b], 0, f)),
            pl.BlockSpec((1, _D, _FF), lambda b, be, f=f: (be[b], 0, f)),
            pl.BlockSpec((1, _FF, _D), lambda b, be, f=f: (be[b], f, 0)),
        ]
        if f == 0:
            y = pl.pallas_call(
                _gemm_body_first,
                grid_spec=pltpu.PrefetchScalarGridSpec(
                    num_scalar_prefetch=1,
                    grid=(nb,),
                    in_specs=row_specs + w_specs,
                    out_specs=out_spec,
                ),
                out_shape=out_shape,
            )(block_expert, x_sorted, w128, wg, wu, wd)
        else:
            y = pl.pallas_call(
                _gemm_body_acc,
                grid_spec=pltpu.PrefetchScalarGridSpec(
                    num_scalar_prefetch=1,
                    grid=(nb,),
                    in_specs=[out_spec] + row_specs + w_specs,
                    out_specs=out_spec,
                ),
                out_shape=out_shape,
                input_output_aliases={1: 0},
            )(block_expert, y, x_sorted, w128, wg, wu, wd)
    return y


def _sc_combine(pos3, yw, n):
    info = plsc.get_sparse_core_info()
    nw = info.num_cores * info.num_subcores
    n_ch, chk = pos3.shape[1], pos3.shape[2]
    ch = chk // _K
    per_w = n_ch * ch
    mesh = plsc.VectorSubcoreMesh(core_axis_name="c", subcore_axis_name="s")

    @functools.partial(
        pl.kernel,
        mesh=mesh,
        out_type=jax.ShapeDtypeStruct((n, _D), jnp.float32),
        scratch_types=[
            pltpu.VMEM((chk,), jnp.int32),
            pltpu.VMEM((chk,), jnp.int32),
            pltpu.VMEM((chk, _D), jnp.float32),
            pltpu.VMEM((chk, _D), jnp.float32),
            pltpu.VMEM((ch, _D), jnp.float32),
            pltpu.SemaphoreType.DMA,
            pltpu.SemaphoreType.DMA,
        ],
    )
    def k(pos_hbm, yw_hbm, out_hbm, i0, i1, r0, r1, out_v, g0, g1):
        wid = lax.axis_index("s") * info.num_cores + lax.axis_index("c")
        base = wid * per_w
        ibuf, rbuf, gsem = [i0, i1], [r0, r1], [g0, g1]

        def fire_gather(c, s):
            pltpu.sync_copy(pos_hbm.at[wid, c], ibuf[s])
            return pltpu.async_copy(yw_hbm.at[ibuf[s]], rbuf[s], gsem[s])

        def add_store(c, s):
            rv = rbuf[s]

            def row(j, _):
                for t in range(_D // 16):
                    sl = pl.ds(t * 16, 16)
                    out_v[j, sl] = rv[_K * j, sl] + rv[_K * j + 1, sl]
                return 0

            lax.fori_loop(0, ch, row, 0)
            pltpu.sync_copy(out_v, out_hbm.at[pl.ds(base + c * ch, ch)])

        pend = [fire_gather(0, 0), None]
        for c in range(n_ch):
            s = c % 2
            o = (c + 1) % 2
            if c + 1 < n_ch:
                pend[o] = fire_gather(c + 1, o)
            pend[s].wait()
            add_store(c, s)

    return k(pos3, yw)


def kernel(input, W_router, W_gate, W_up, W_down):
    b, s, d = input.shape
    n = b * s
    x = input.reshape(n, d)
    wr_pad = jnp.zeros((d, _LANES), jnp.float32).at[:, :_E].set(W_router)
    topi, probs = _router(x, wr_pad)
    row_w, block_expert, dest = _plan(topi, probs, n)
    dest3 = dest.reshape(32, -1, 32)
    x_sorted = _sc_dispatch(dest3, x, row_w.shape[0])
    w128 = jnp.broadcast_to(row_w[:, None], (row_w.shape[0], _LANES))
    yw = _gemm(block_expert, x_sorted, w128, W_gate, W_up, W_down)
    out = _sc_combine(dest3, yw, n)
    return out.reshape(b, s, d)


# speedup vs baseline: 1.4980x; 1.0353x over previous
import functools

import jax
import jax.numpy as jnp
from jax import lax
from jax.experimental import pallas as pl
from jax.experimental.pallas import tpu as pltpu
from jax.experimental.pallas import tpu_sc as plsc

_E = 8
_K = 2
_D = 1024
_F = 4096
_BLK = 256
_FF = 2048
_NF = _F // _FF
_LANES = 128


def _router_body(x_ref, wr_ref, idx_ref, prob_ref):
    x = x_ref[...]
    wr = wr_ref[...]
    logits = jnp.dot(x, wr, preferred_element_type=jnp.float32)
    lane = lax.broadcasted_iota(jnp.int32, logits.shape, 1)
    neg = jnp.float32(-1e30)
    logits = jnp.where(lane < _E, logits, neg)
    m1 = jnp.max(logits, axis=1, keepdims=True)
    a1 = jnp.min(jnp.where(logits == m1, lane, _LANES), axis=1, keepdims=True)
    l2 = jnp.where(lane == a1, neg, logits)
    m2 = jnp.max(l2, axis=1, keepdims=True)
    a2 = jnp.min(jnp.where(l2 == m2, lane, _LANES), axis=1, keepdims=True)
    e2 = jnp.exp(m2 - m1)
    p1 = 1.0 / (1.0 + e2)
    p2 = e2 / (1.0 + e2)
    idx_ref[...] = jnp.where(lane == 0, a1, jnp.where(lane == 1, a2, 0))
    prob_ref[...] = jnp.where(lane == 0, p1, jnp.where(lane == 1, p2, 0.0))


def _router(x, wr_pad):
    n = x.shape[0]
    t = 512
    idx, prob = pl.pallas_call(
        _router_body,
        grid=(n // t,),
        in_specs=[
            pl.BlockSpec((t, _D), lambda i: (i, 0)),
            pl.BlockSpec((_D, _LANES), lambda i: (0, 0)),
        ],
        out_specs=[
            pl.BlockSpec((t, _LANES), lambda i: (i, 0)),
            pl.BlockSpec((t, _LANES), lambda i: (i, 0)),
        ],
        out_shape=[
            jax.ShapeDtypeStruct((n, _LANES), jnp.int32),
            jax.ShapeDtypeStruct((n, _LANES), jnp.float32),
        ],
    )(x, wr_pad)
    return idx[:, :_K], prob[:, :_K]


def _plan(topi, probs, n):
    p_total = _K * n + _E * _BLK
    e_pairs = topi.reshape(-1)
    w_pairs = probs.reshape(-1)
    oh = (e_pairs[:, None] == jnp.arange(_E, dtype=jnp.int32)[None, :]).astype(jnp.int32)
    counts = jnp.sum(oh, axis=0)
    rank = jnp.sum((jnp.cumsum(oh, axis=0) - oh) * oh, axis=1)
    padded = ((counts + _BLK - 1) // _BLK) * _BLK
    ends = jnp.cumsum(padded)
    starts = ends - padded
    dest = starts[e_pairs] + rank
    bstart = jnp.arange(p_total // _BLK, dtype=jnp.int32) * _BLK
    block_expert = jnp.minimum(
        jnp.sum((bstart[:, None] >= ends[None, :]).astype(jnp.int32), axis=1),
        _E - 1).astype(jnp.int32)
    block_used = ((bstart - starts[block_expert])
                  < counts[block_expert]).astype(jnp.int32)
    return block_expert, block_used, dest.astype(jnp.int32)


def _sc_dispatch(dest3, x_flat, p_total):
    info = plsc.get_sparse_core_info()
    nw = info.num_cores * info.num_subcores
    n_ch, ch = dest3.shape[1], dest3.shape[2]
    per_w = n_ch * ch
    mesh = plsc.VectorSubcoreMesh(core_axis_name="c", subcore_axis_name="s")

    @functools.partial(
        pl.kernel,
        mesh=mesh,
        out_type=jax.ShapeDtypeStruct((p_total, _D), jnp.float32),
        scratch_types=[
            pltpu.VMEM((ch,), jnp.int32),
            pltpu.VMEM((ch,), jnp.int32),
            pltpu.VMEM((ch,), jnp.int32),
            pltpu.VMEM((ch,), jnp.int32),
            pltpu.VMEM((ch, _D), jnp.float32),
            pltpu.VMEM((ch, _D), jnp.float32),
            pltpu.SemaphoreType.DMA,
            pltpu.SemaphoreType.DMA,
            pltpu.SemaphoreType.DMA,
            pltpu.SemaphoreType.DMA,
        ],
    )
    def k(dest_hbm, x_hbm, out_hbm, i0, i1, d0, d1, r0, r1, g0, g1, s0, s1):
        wid = lax.axis_index("s") * info.num_cores + lax.axis_index("c")
        tbase = wid * (per_w // _K)
        ibuf, dbuf, rbuf = [i0, i1], [d0, d1], [r0, r1]
        gsem, ssem = [g0, g1], [s0, s1]
        iota = lax.broadcasted_iota(jnp.int32, (16,), 0)

        def fire_gather(c, s):
            for t in range(ch // 16):
                ibuf[s][pl.ds(16 * t, 16)] = (
                    tbase + c * (ch // _K) + ((iota + 16 * t) >> 1))
            return pltpu.async_copy(x_hbm.at[ibuf[s]], rbuf[s], gsem[s])

        pend_g = [fire_gather(0, 0), None]
        pend_s = [None, None]
        for c in range(n_ch):
            s = c % 2
            o = (c + 1) % 2
            if c + 1 < n_ch:
                if pend_s[o] is not None:
                    pend_s[o].wait()
                pend_g[o] = fire_gather(c + 1, o)
            pend_g[s].wait()
            pltpu.sync_copy(dest_hbm.at[wid, c], dbuf[s])
            pend_s[s] = pltpu.async_copy(rbuf[s], out_hbm.at[dbuf[s]], ssem[s])
        for s in range(2):
            if pend_s[s] is not None:
                pend_s[s].wait()

    return k(dest3, x_flat)


def _swiglu_part(x_ref, wg_ref, wu_ref, wd_ref):
    x = x_ref[...].astype(jnp.bfloat16)
    g = jnp.dot(x, wg_ref[0].astype(jnp.bfloat16),
                preferred_element_type=jnp.float32)
    u = jnp.dot(x, wu_ref[0].astype(jnp.bfloat16),
                preferred_element_type=jnp.float32)
    h = (g * jax.nn.sigmoid(g) * u).astype(jnp.bfloat16)
    return jnp.dot(h, wd_ref[0].astype(jnp.bfloat16),
                   preferred_element_type=jnp.float32)


def _gemm_body_first(be_ref, bu_ref, x_ref, wg_ref, wu_ref, wd_ref, y_ref):
    @pl.when(bu_ref[pl.program_id(0)] != 0)
    def _():
        y_ref[...] = _swiglu_part(x_ref, wg_ref, wu_ref, wd_ref)


def _gemm_body_acc(be_ref, bu_ref, y_in_ref, x_ref, wg_ref, wu_ref, wd_ref,
                   y_ref):
    @pl.when(bu_ref[pl.program_id(0)] != 0)
    def _():
        y_ref[...] = y_in_ref[...] + _swiglu_part(x_ref, wg_ref, wu_ref,
                                                  wd_ref)


def _gemm(block_expert, block_used, x_sorted, wg, wu, wd):
    p_total = x_sorted.shape[0]
    nb = p_total // _BLK
    x_spec = pl.BlockSpec((_BLK, _D), lambda b, be, bu: (b, 0))
    out_spec = pl.BlockSpec((_BLK, _D), lambda b, be, bu: (b, 0))
    out_shape = jax.ShapeDtypeStruct((p_total, _D), jnp.float32)
    y = None
    for f in range(_NF):
        w_specs = [
            pl.BlockSpec((1, _D, _FF), lambda b, be, bu, f=f: (be[b], 0, f)),
            pl.BlockSpec((1, _D, _FF), lambda b, be, bu, f=f: (be[b], 0, f)),
            pl.BlockSpec((1, _FF, _D), lambda b, be, bu, f=f: (be[b], f, 0)),
        ]
        if f == 0:
            y = pl.pallas_call(
                _gemm_body_first,
                grid_spec=pltpu.PrefetchScalarGridSpec(
                    num_scalar_prefetch=2,
                    grid=(nb,),
                    in_specs=[x_spec] + w_specs,
                    out_specs=out_spec,
                ),
                out_shape=out_shape,
            )(block_expert, block_used, x_sorted, wg, wu, wd)
        else:
            y = pl.pallas_call(
                _gemm_body_acc,
                grid_spec=pltpu.PrefetchScalarGridSpec(
                    num_scalar_prefetch=2,
                    grid=(nb,),
                    in_specs=[out_spec, x_spec] + w_specs,
                    out_specs=out_spec,
                ),
                out_shape=out_shape,
                input_output_aliases={2: 0},
            )(block_expert, block_used, y, x_sorted, wg, wu, wd)
    return y


def _sc_combine(pos3, w16, yw, n):
    info = plsc.get_sparse_core_info()
    nw = info.num_cores * info.num_subcores
    n_ch, chk = pos3.shape[1], pos3.shape[2]
    ch = chk // _K
    per_w = n_ch * ch
    mesh = plsc.VectorSubcoreMesh(core_axis_name="c", subcore_axis_name="s")

    @functools.partial(
        pl.kernel,
        mesh=mesh,
        out_type=jax.ShapeDtypeStruct((n, _D), jnp.float32),
        scratch_types=[
            pltpu.VMEM((chk,), jnp.int32),
            pltpu.VMEM((chk,), jnp.int32),
            pltpu.VMEM((chk, 16), jnp.float32),
            pltpu.VMEM((chk, 16), jnp.float32),
            pltpu.VMEM((chk, _D), jnp.float32),
            pltpu.VMEM((chk, _D), jnp.float32),
            pltpu.VMEM((ch, _D), jnp.float32),
            pltpu.SemaphoreType.DMA,
            pltpu.SemaphoreType.DMA,
        ],
    )
    def k(pos_hbm, w_hbm, yw_hbm, out_hbm, i0, i1, w0, w1, r0, r1, out_v,
          g0, g1):
        wid = lax.axis_index("s") * info.num_cores + lax.axis_index("c")
        base = wid * per_w
        ibuf, wbuf, rbuf, gsem = [i0, i1], [w0, w1], [r0, r1], [g0, g1]

        def fire_gather(c, s):
            pltpu.sync_copy(pos_hbm.at[wid, c], ibuf[s])
            pltpu.sync_copy(w_hbm.at[wid, c], wbuf[s])
            return pltpu.async_copy(yw_hbm.at[ibuf[s]], rbuf[s], gsem[s])

        def add_store(c, s):
            rv, wv = rbuf[s], wbuf[s]

            def row(j, _):
                wa = wv[_K * j, :]
                wb = wv[_K * j + 1, :]
                for t in range(_D // 16):
                    sl = pl.ds(t * 16, 16)
                    out_v[j, sl] = wa * rv[_K * j, sl] + wb * rv[_K * j + 1, sl]
                return 0

            lax.fori_loop(0, ch, row, 0)
            pltpu.sync_copy(out_v, out_hbm.at[pl.ds(base + c * ch, ch)])

        pend = [fire_gather(0, 0), None]
        for c in range(n_ch):
            s = c % 2
            o = (c + 1) % 2
            if c + 1 < n_ch:
                pend[o] = fire_gather(c + 1, o)
            pend[s].wait()
            add_store(c, s)

    return k(pos3, w16, yw)


def kernel(input, W_router, W_gate, W_up, W_down):
    b, s, d = input.shape
    n = b * s
    x = input.reshape(n, d)
    wr_pad = jnp.zeros((d, _LANES), jnp.float32).at[:, :_E].set(W_router)
    topi, probs = _router(x, wr_pad)
    block_expert, block_used, dest = _plan(topi, probs, n)
    dest3 = dest.reshape(32, -1, 32)
    p_total = _K * n + _E * _BLK
    x_sorted = _sc_dispatch(dest3, x, p_total)
    yw = _gemm(block_expert, block_used, x_sorted, W_gate, W_up, W_down)
    w16 = jnp.broadcast_to(
        probs.reshape(32, -1, 32)[..., None], dest3.shape + (16,))
    out = _sc_combine(dest3, w16, yw, n)
    return out.reshape(b, s, d)


# speedup vs baseline: 1.5076x; 1.0064x over previous
import functools

import jax
import jax.numpy as jnp
from jax import lax
from jax.experimental import pallas as pl
from jax.experimental.pallas import tpu as pltpu
from jax.experimental.pallas import tpu_sc as plsc

_E = 8
_K = 2
_D = 1024
_F = 4096
_BLK = 256
_FF = 2048
_NF = _F // _FF
_LANES = 128


def _router_body(x_ref, wr_ref, idx_ref, prob_ref):
    x = x_ref[...]
    wr = wr_ref[...]
    logits = jnp.dot(x, wr, preferred_element_type=jnp.float32)
    lane = lax.broadcasted_iota(jnp.int32, logits.shape, 1)
    neg = jnp.float32(-1e30)
    logits = jnp.where(lane < _E, logits, neg)
    m1 = jnp.max(logits, axis=1, keepdims=True)
    a1 = jnp.min(jnp.where(logits == m1, lane, _LANES), axis=1, keepdims=True)
    l2 = jnp.where(lane == a1, neg, logits)
    m2 = jnp.max(l2, axis=1, keepdims=True)
    a2 = jnp.min(jnp.where(l2 == m2, lane, _LANES), axis=1, keepdims=True)
    e2 = jnp.exp(m2 - m1)
    p1 = 1.0 / (1.0 + e2)
    p2 = e2 / (1.0 + e2)
    idx_ref[...] = jnp.where(lane == 0, a1, jnp.where(lane == 1, a2, 0))
    prob_ref[...] = jnp.where(lane == 0, p1, jnp.where(lane == 1, p2, 0.0))


def _router(x, wr_pad):
    n = x.shape[0]
    t = 512
    idx, prob = pl.pallas_call(
        _router_body,
        grid=(n // t,),
        in_specs=[
            pl.BlockSpec((t, _D), lambda i: (i, 0)),
            pl.BlockSpec((_D, _LANES), lambda i: (0, 0)),
        ],
        out_specs=[
            pl.BlockSpec((t, _LANES), lambda i: (i, 0)),
            pl.BlockSpec((t, _LANES), lambda i: (i, 0)),
        ],
        out_shape=[
            jax.ShapeDtypeStruct((n, _LANES), jnp.int32),
            jax.ShapeDtypeStruct((n, _LANES), jnp.float32),
        ],
    )(x, wr_pad)
    return idx[:, :_K], prob[:, :_K]


def _plan(topi, probs, n):
    p_total = _K * n + _E * _BLK
    e_pairs = topi.reshape(-1)
    w_pairs = probs.reshape(-1)
    oh = (e_pairs[:, None] == jnp.arange(_E, dtype=jnp.int32)[None, :]).astype(jnp.int32)
    counts = jnp.sum(oh, axis=0)
    rank = jnp.sum((jnp.cumsum(oh, axis=0) - oh) * oh, axis=1)
    padded = ((counts + _BLK - 1) // _BLK) * _BLK
    ends = jnp.cumsum(padded)
    starts = ends - padded
    dest = starts[e_pairs] + rank
    bstart = jnp.arange(p_total // _BLK, dtype=jnp.int32) * _BLK
    block_expert = jnp.minimum(
        jnp.sum((bstart[:, None] >= ends[None, :]).astype(jnp.int32), axis=1),
        _E - 1).astype(jnp.int32)
    block_used = ((bstart - starts[block_expert])
                  < counts[block_expert]).astype(jnp.int32)
    return block_expert, block_used, dest.astype(jnp.int32)


def _sc_dispatch(dest3, x_flat, p_total):
    info = plsc.get_sparse_core_info()
    nw = info.num_cores * info.num_subcores
    n_ch, ch = dest3.shape[1], dest3.shape[2]
    per_w = n_ch * ch
    mesh = plsc.VectorSubcoreMesh(core_axis_name="c", subcore_axis_name="s")

    @functools.partial(
        pl.kernel,
        mesh=mesh,
        out_type=jax.ShapeDtypeStruct((p_total, _D), jnp.float32),
        scratch_types=[
            pltpu.VMEM((ch,), jnp.int32),
            pltpu.VMEM((ch,), jnp.int32),
            pltpu.VMEM((ch,), jnp.int32),
            pltpu.VMEM((ch,), jnp.int32),
            pltpu.VMEM((ch, _D), jnp.float32),
            pltpu.VMEM((ch, _D), jnp.float32),
            pltpu.SemaphoreType.DMA,
            pltpu.SemaphoreType.DMA,
            pltpu.SemaphoreType.DMA,
            pltpu.SemaphoreType.DMA,
        ],
    )
    def k(dest_hbm, x_hbm, out_hbm, i0, i1, d0, d1, r0, r1, g0, g1, s0, s1):
        wid = lax.axis_index("s") * info.num_cores + lax.axis_index("c")
        tbase = wid * (per_w // _K)
        ibuf, dbuf, rbuf = [i0, i1], [d0, d1], [r0, r1]
        gsem, ssem = [g0, g1], [s0, s1]
        iota = lax.broadcasted_iota(jnp.int32, (16,), 0)

        def fire_gather(c, s):
            for t in range(ch // 16):
                ibuf[s][pl.ds(16 * t, 16)] = (
                    tbase + c * (ch // _K) + ((iota + 16 * t) >> 1))
            return pltpu.async_copy(x_hbm.at[ibuf[s]], rbuf[s], gsem[s])

        pend_g = [fire_gather(0, 0), None]
        pend_s = [None, None]
        for c in range(n_ch):
            s = c % 2
            o = (c + 1) % 2
            if c + 1 < n_ch:
                if pend_s[o] is not None:
                    pend_s[o].wait()
                pend_g[o] = fire_gather(c + 1, o)
            pend_g[s].wait()
            pltpu.sync_copy(dest_hbm.at[wid, c], dbuf[s])
            pend_s[s] = pltpu.async_copy(rbuf[s], out_hbm.at[dbuf[s]], ssem[s])
        for s in range(2):
            if pend_s[s] is not None:
                pend_s[s].wait()

    return k(dest3, x_flat)


def _swiglu_part(x_ref, wg_ref, wu_ref, wd_ref):
    x = x_ref[...].astype(jnp.bfloat16)
    g = jnp.dot(x, wg_ref[0].astype(jnp.bfloat16),
                preferred_element_type=jnp.float32)
    u = jnp.dot(x, wu_ref[0].astype(jnp.bfloat16),
                preferred_element_type=jnp.float32)
    h = (g * jax.nn.sigmoid(g) * u).astype(jnp.bfloat16)
    return jnp.dot(h, wd_ref[0].astype(jnp.bfloat16),
                   preferred_element_type=jnp.float32)


def _gemm_body_first(be_ref, bu_ref, x_ref, wg_ref, wu_ref, wd_ref, y_ref):
    @pl.when(bu_ref[pl.program_id(0)] != 0)
    def _():
        y_ref[...] = _swiglu_part(x_ref, wg_ref, wu_ref, wd_ref)


def _gemm_body_acc(be_ref, bu_ref, y_in_ref, x_ref, wg_ref, wu_ref, wd_ref,
                   y_ref):
    @pl.when(bu_ref[pl.program_id(0)] != 0)
    def _():
        y_ref[...] = y_in_ref[...] + _swiglu_part(x_ref, wg_ref, wu_ref,
                                                  wd_ref)


def _gemm(block_expert, block_used, x_sorted, wg, wu, wd):
    p_total = x_sorted.shape[0]
    nb = p_total // _BLK
    x_spec = pl.BlockSpec((_BLK, _D), lambda b, be, bu: (b, 0))
    out_spec = pl.BlockSpec((_BLK, _D), lambda b, be, bu: (b, 0))
    out_shape = jax.ShapeDtypeStruct((p_total, _D), jnp.float32)
    y = None
    for f in range(_NF):
        w_specs = [
            pl.BlockSpec((1, _D, _FF), lambda b, be, bu, f=f: (be[b], 0, f)),
            pl.BlockSpec((1, _D, _FF), lambda b, be, bu, f=f: (be[b], 0, f)),
            pl.BlockSpec((1, _FF, _D), lambda b, be, bu, f=f: (be[b], f, 0)),
        ]
        if f == 0:
            y = pl.pallas_call(
                _gemm_body_first,
                grid_spec=pltpu.PrefetchScalarGridSpec(
                    num_scalar_prefetch=2,
                    grid=(nb,),
                    in_specs=[x_spec] + w_specs,
                    out_specs=out_spec,
                ),
                out_shape=out_shape,
            )(block_expert, block_used, x_sorted, wg, wu, wd)
        else:
            y = pl.pallas_call(
                _gemm_body_acc,
                grid_spec=pltpu.PrefetchScalarGridSpec(
                    num_scalar_prefetch=2,
                    grid=(nb,),
                    in_specs=[out_spec, x_spec] + w_specs,
                    out_specs=out_spec,
                ),
                out_shape=out_shape,
                input_output_aliases={2: 0},
            )(block_expert, block_used, y, x_sorted, wg, wu, wd)
    return y


def _sc_combine(pos3, w16, yw, n):
    info = plsc.get_sparse_core_info()
    nw = info.num_cores * info.num_subcores
    n_ch, chk = pos3.shape[1], pos3.shape[2]
    ch = chk // _K
    per_w = n_ch * ch
    mesh = plsc.VectorSubcoreMesh(core_axis_name="c", subcore_axis_name="s")

    @functools.partial(
        pl.kernel,
        mesh=mesh,
        out_type=jax.ShapeDtypeStruct((n, _D), jnp.float32),
        scratch_types=[
            pltpu.VMEM((chk,), jnp.int32),
            pltpu.VMEM((chk,), jnp.int32),
            pltpu.VMEM((chk, 16), jnp.float32),
            pltpu.VMEM((chk, 16), jnp.float32),
            pltpu.VMEM((chk, _D), jnp.float32),
            pltpu.VMEM((chk, _D), jnp.float32),
            pltpu.VMEM((ch, _D), jnp.float32),
            pltpu.VMEM((ch, _D), jnp.float32),
            pltpu.SemaphoreType.DMA,
            pltpu.SemaphoreType.DMA,
            pltpu.SemaphoreType.DMA,
            pltpu.SemaphoreType.DMA,
        ],
    )
    def k(pos_hbm, w_hbm, yw_hbm, out_hbm, i0, i1, w0, w1, r0, r1, o0, o1,
          g0, g1, s0, s1):
        wid = lax.axis_index("s") * info.num_cores + lax.axis_index("c")
        base = wid * per_w
        ibuf, wbuf, rbuf, gsem = [i0, i1], [w0, w1], [r0, r1], [g0, g1]
        obuf, ssem = [o0, o1], [s0, s1]

        def fire_gather(c, s):
            pltpu.sync_copy(pos_hbm.at[wid, c], ibuf[s])
            pltpu.sync_copy(w_hbm.at[wid, c], wbuf[s])
            return pltpu.async_copy(yw_hbm.at[ibuf[s]], rbuf[s], gsem[s])

        def add_store(c, s):
            rv, wv, ov = rbuf[s], wbuf[s], obuf[s]

            def row(j, _):
                wa = wv[_K * j, :]
                wb = wv[_K * j + 1, :]
                for t in range(_D // 16):
                    sl = pl.ds(t * 16, 16)
                    ov[j, sl] = wa * rv[_K * j, sl] + wb * rv[_K * j + 1, sl]
                return 0

            lax.fori_loop(0, ch, row, 0)
            return pltpu.async_copy(ov, out_hbm.at[pl.ds(base + c * ch, ch)],
                                    ssem[s])

        pend = [fire_gather(0, 0), None]
        pend_s = [None, None]
        for c in range(n_ch):
            s = c % 2
            o = (c + 1) % 2
            if c + 1 < n_ch:
                pend[o] = fire_gather(c + 1, o)
            pend[s].wait()
            if pend_s[s] is not None:
                pend_s[s].wait()
            pend_s[s] = add_store(c, s)
        for s in range(2):
            if pend_s[s] is not None:
                pend_s[s].wait()

    return k(pos3, w16, yw)


def kernel(input, W_router, W_gate, W_up, W_down):
    b, s, d = input.shape
    n = b * s
    x = input.reshape(n, d)
    wr_pad = jnp.zeros((d, _LANES), jnp.float32).at[:, :_E].set(W_router)
    topi, probs = _router(x, wr_pad)
    block_expert, block_used, dest = _plan(topi, probs, n)
    dest3 = dest.reshape(32, -1, 32)
    p_total = _K * n + _E * _BLK
    x_sorted = _sc_dispatch(dest3, x, p_total)
    yw = _gemm(block_expert, block_used, x_sorted, W_gate, W_up, W_down)
    w16 = jnp.broadcast_to(
        probs.reshape(32, -1, 32)[..., None], dest3.shape + (16,))
    out = _sc_combine(dest3, w16, yw, n)
    return out.reshape(b, s, d)
